# Initial kernel scaffold; baseline (speedup 1.0000x reference)
import jax
import jax.numpy as jnp
from jax.experimental import pallas as pl


def kernel(x, edge_index, Wl0, bl0, Wr0, Wl1, bl1, Wr1, Wl2, bl2, Wr2):
    raise NotImplementedError("write your pallas kernel here")


# speedup vs baseline: 6.0254x; 6.0254x over previous
import functools

import jax
import jax.numpy as jnp
from jax import lax
from jax.experimental import pallas as pl
from jax.experimental.pallas import tpu as pltpu
from jax.experimental.pallas import tpu_sc as plsc

N = 10000
E = 320000
D = 128
H = 128
C = 64

NC, NS = 2, 16
NW = NC * NS
EC = 128
NEC = E // EC
NLOOP = -(-NEC // NW)
RC = 80
NRC = N // RC
NRLOOP = -(-NRC // NS)
DW = 16

_F32 = jnp.float32


def _make_sc_agg(d, with_deg):
    mesh = plsc.VectorSubcoreMesh(core_axis_name="c", subcore_axis_name="s")
    out_type = [jax.ShapeDtypeStruct((NC, N, d), _F32)]
    scratch = [
        pltpu.VMEM_SHARED((N, d), _F32),
        pltpu.VMEM((EC,), jnp.int32),
        pltpu.VMEM((EC,), jnp.int32),
        pltpu.VMEM((EC, d), _F32),
        pltpu.SemaphoreType.DMA,
    ]
    if with_deg:
        out_type.append(jax.ShapeDtypeStruct((NC, N, DW), _F32))
        scratch += [
            pltpu.VMEM_SHARED((N, DW), _F32),
            pltpu.VMEM((EC, DW), _F32),
        ]

    @functools.partial(pl.kernel, out_type=tuple(out_type), mesh=mesh,
                       scratch_types=scratch,
                       compiler_params=pltpu.CompilerParams(
                           use_tc_tiling_on_sc=False))
    def k(u_hbm, src_hbm, dst_hbm, zrows_hbm, zvec_hbm, ones_hbm, *refs):
        if with_deg:
            out, degout, acc, srcv, dstv, rows, sem, dega, onesv = refs
        else:
            out, acc, srcv, dstv, rows, sem = refs
        cid = lax.axis_index("c")
        sid = lax.axis_index("s")
        wid = sid * NC + cid

        @pl.loop(0, NRLOOP)
        def _zero(kk):
            c = sid + NS * kk

            @pl.when(c < NRC)
            def _():
                pltpu.sync_copy(zrows_hbm, acc.at[pl.ds(c * RC, RC)])
                if with_deg:
                    pltpu.sync_copy(zvec_hbm, dega.at[pl.ds(c * RC, RC)])

        if with_deg:
            pltpu.sync_copy(ones_hbm, onesv)
        plsc.subcore_barrier()

        @pl.loop(0, NLOOP)
        def _edges(kk):
            c = wid + NW * kk

            @pl.when(c < NEC)
            def _():
                pltpu.sync_copy(src_hbm.at[pl.ds(c * EC, EC)], srcv)
                pltpu.sync_copy(dst_hbm.at[pl.ds(c * EC, EC)], dstv)
                pltpu.async_copy(u_hbm.at[srcv], rows, sem).wait()
                pltpu.sync_copy(rows, acc.at[dstv], add=True)
                if with_deg:
                    pltpu.sync_copy(onesv, dega.at[dstv], add=True)

        plsc.subcore_barrier()

        @pl.loop(0, NRLOOP)
        def _wout(kk):
            c = sid + NS * kk

            @pl.when(c < NRC)
            def _():
                pltpu.sync_copy(acc.at[pl.ds(c * RC, RC)],
                                out.at[cid, pl.ds(c * RC, RC)])
                if with_deg:
                    pltpu.sync_copy(dega.at[pl.ds(c * RC, RC)],
                                    degout.at[cid, pl.ds(c * RC, RC)])

    return k


_sc_agg_deg = _make_sc_agg(H, True)
_sc_agg_h = _make_sc_agg(H, False)
_sc_agg_c = _make_sc_agg(C, False)

BN = 1000


def _dot_t(a, w):
    return lax.dot_general(a, w, (((1,), (1,)), ((), ())),
                           precision=lax.Precision.HIGHEST,
                           preferred_element_type=_F32)


def _tc_first_body(x_ref, wl_ref, wr_ref, bl_ref, u_ref, v_ref):
    x = x_ref[...]
    u_ref[...] = _dot_t(x, wl_ref[...])
    v_ref[...] = _dot_t(x, wr_ref[...]) + bl_ref[...]


def _tc_first(x, wl, wr, bl):
    dout = wl.shape[0]
    return pl.pallas_call(
        _tc_first_body,
        grid=(N // BN,),
        in_specs=[
            pl.BlockSpec((BN, x.shape[1]), lambda i: (i, 0)),
            pl.BlockSpec(wl.shape, lambda i: (0, 0)),
            pl.BlockSpec(wr.shape, lambda i: (0, 0)),
            pl.BlockSpec((1, dout), lambda i: (0, 0)),
        ],
        out_specs=[
            pl.BlockSpec((BN, dout), lambda i: (i, 0)),
            pl.BlockSpec((BN, dout), lambda i: (i, 0)),
        ],
        out_shape=[
            jax.ShapeDtypeStruct((N, dout), _F32),
            jax.ShapeDtypeStruct((N, dout), _F32),
        ],
    )(x, wl, wr, bl)


def _tc_mid_body(sa_ref, sb_ref, da_ref, db_ref, v_ref, wl_ref, wr_ref,
                 bl_ref, u_ref, vn_ref):
    deg = da_ref[...] + db_ref[...]
    rdeg = 1.0 / jnp.maximum(deg, 1.0)
    h = jnp.maximum((sa_ref[...] + sb_ref[...]) * rdeg + v_ref[...], 0.0)
    u_ref[...] = _dot_t(h, wl_ref[...])
    vn_ref[...] = _dot_t(h, wr_ref[...]) + bl_ref[...]


def _tc_mid(sa, sb, da, db, v, wl, wr, bl):
    din = sa.shape[1]
    dout = wl.shape[0]
    return pl.pallas_call(
        _tc_mid_body,
        grid=(N // BN,),
        in_specs=[
            pl.BlockSpec((BN, din), lambda i: (i, 0)),
            pl.BlockSpec((BN, din), lambda i: (i, 0)),
            pl.BlockSpec((BN, 1), lambda i: (i, 0)),
            pl.BlockSpec((BN, 1), lambda i: (i, 0)),
            pl.BlockSpec((BN, din), lambda i: (i, 0)),
            pl.BlockSpec(wl.shape, lambda i: (0, 0)),
            pl.BlockSpec(wr.shape, lambda i: (0, 0)),
            pl.BlockSpec((1, dout), lambda i: (0, 0)),
        ],
        out_specs=[
            pl.BlockSpec((BN, dout), lambda i: (i, 0)),
            pl.BlockSpec((BN, dout), lambda i: (i, 0)),
        ],
        out_shape=[
            jax.ShapeDtypeStruct((N, dout), _F32),
            jax.ShapeDtypeStruct((N, dout), _F32),
        ],
    )(sa, sb, da, db, v, wl, wr, bl)


def _tc_final_body(sa_ref, sb_ref, da_ref, db_ref, v_ref, o_ref):
    deg = da_ref[...] + db_ref[...]
    rdeg = 1.0 / jnp.maximum(deg, 1.0)
    o_ref[...] = (sa_ref[...] + sb_ref[...]) * rdeg + v_ref[...]


def _tc_final(sa, sb, da, db, v):
    dout = sa.shape[1]
    return pl.pallas_call(
        _tc_final_body,
        grid=(N // BN,),
        in_specs=[
            pl.BlockSpec((BN, dout), lambda i: (i, 0)),
            pl.BlockSpec((BN, dout), lambda i: (i, 0)),
            pl.BlockSpec((BN, 1), lambda i: (i, 0)),
            pl.BlockSpec((BN, 1), lambda i: (i, 0)),
            pl.BlockSpec((BN, dout), lambda i: (i, 0)),
        ],
        out_specs=pl.BlockSpec((BN, dout), lambda i: (i, 0)),
        out_shape=jax.ShapeDtypeStruct((N, dout), _F32),
    )(sa, sb, da, db, v)


def kernel(x, edge_index, Wl0, bl0, Wr0, Wl1, bl1, Wr1, Wl2, bl2, Wr2):
    src = edge_index[0]
    dst = edge_index[1]
    zrows_h = jnp.zeros((RC, H), _F32)
    zrows_c = jnp.zeros((RC, C), _F32)
    zvec = jnp.zeros((RC, DW), _F32)
    ones_e = jnp.ones((EC, DW), _F32)

    u0, v0 = _tc_first(x, Wl0, Wr0, bl0.reshape(1, -1))
    s0, deg2 = _sc_agg_deg(u0, src, dst, zrows_h, zvec, ones_e)
    da, db = deg2[0, :, :1], deg2[1, :, :1]
    u1, v1 = _tc_mid(s0[0], s0[1], da, db, v0, Wl1, Wr1, bl1.reshape(1, -1))
    (s1,) = _sc_agg_h(u1, src, dst, zrows_h, zvec, ones_e)
    u2, v2 = _tc_mid(s1[0], s1[1], da, db, v1, Wl2, Wr2, bl2.reshape(1, -1))
    (s2,) = _sc_agg_c(u2, src, dst, zrows_c, zvec, ones_e)
    return _tc_final(s2[0], s2[1], da, db, v2)


# speedup vs baseline: 9.5644x; 1.5874x over previous
import functools

import jax
import jax.numpy as jnp
from jax import lax
from jax.experimental import pallas as pl
from jax.experimental.pallas import tpu as pltpu
from jax.experimental.pallas import tpu_sc as plsc

N = 10000
E = 320000
D = 128
H = 128
C = 64

NC, NS = 2, 16
NW = NC * NS
EW = E // NW
KB = 125
KI = EW // KB
RC = 80
NRC = N // RC
NRLOOP = -(-NRC // NS)
DW = 16

_F32 = jnp.float32


def _make_sc_agg(d, with_deg):
    mesh = plsc.VectorSubcoreMesh(core_axis_name="c", subcore_axis_name="s")
    out_type = [jax.ShapeDtypeStruct((NC, N, d), _F32)]
    scratch = [
        pltpu.VMEM_SHARED((N, d), _F32),
        pltpu.VMEM((2, KB), jnp.int32),
        pltpu.VMEM((2, KB), jnp.int32),
        pltpu.VMEM((KB, d), _F32),
        pltpu.VMEM((KB, d), _F32),
        pltpu.SemaphoreType.DMA,
        pltpu.SemaphoreType.DMA,
        pltpu.SemaphoreType.DMA,
        pltpu.SemaphoreType.DMA,
    ]
    if with_deg:
        out_type.append(jax.ShapeDtypeStruct((NC, N, DW), _F32))
        scratch += [
            pltpu.VMEM_SHARED((N, DW), _F32),
            pltpu.VMEM((KB, DW), _F32),
        ]

    @functools.partial(pl.kernel, out_type=tuple(out_type), mesh=mesh,
                       scratch_types=scratch,
                       compiler_params=pltpu.CompilerParams(
                           use_tc_tiling_on_sc=False))
    def k(u_hbm, eidx_hbm, zrows_hbm, zvec_hbm, ones_hbm, *refs):
        if with_deg:
            (out, degout, acc, idx0, idx1, rows0, rows1, semg0, semg1,
             semi0, semi1, dega, onesv) = refs
        else:
            (out, acc, idx0, idx1, rows0, rows1, semg0, semg1,
             semi0, semi1) = refs
        cid = lax.axis_index("c")
        sid = lax.axis_index("s")
        wid = sid * NC + cid

        if with_deg:
            pltpu.sync_copy(ones_hbm, onesv)

        @pl.loop(0, NRLOOP)
        def _zero(kk):
            c = sid + NS * kk

            @pl.when(c < NRC)
            def _():
                pltpu.sync_copy(zrows_hbm, acc.at[pl.ds(c * RC, RC)])
                if with_deg:
                    pltpu.sync_copy(zvec_hbm, dega.at[pl.ds(c * RC, RC)])

        plsc.subcore_barrier()

        pltpu.sync_copy(eidx_hbm.at[wid, 0], idx0)
        pltpu.async_copy(u_hbm.at[idx0.at[0]], rows0, semg0)
        pltpu.async_copy(eidx_hbm.at[wid, 1], idx1, semi1)

        def _halfstep(kc, idxa, rowsa, semga, semia, idxb, rowsb, semgb, semib):
            pltpu.make_async_copy(u_hbm.at[idxa.at[0]], rowsa, semga).wait()

            @pl.when(kc + 1 < KI)
            def _():
                pltpu.make_async_copy(eidx_hbm.at[wid, 0], idxb, semib).wait()
                pltpu.async_copy(u_hbm.at[idxb.at[0]], rowsb, semgb)

            pltpu.sync_copy(rowsa, acc.at[idxa.at[1]], add=True)
            if with_deg:
                pltpu.sync_copy(onesv, dega.at[idxa.at[1]], add=True)

            @pl.when(kc + 2 < KI)
            def _():
                pltpu.async_copy(eidx_hbm.at[wid, kc + 2], idxa, semia)

        @pl.loop(0, KI // 2)
        def _edges(kk):
            k0 = 2 * kk
            _halfstep(k0, idx0, rows0, semg0, semi0, idx1, rows1, semg1, semi1)
            _halfstep(k0 + 1, idx1, rows1, semg1, semi1, idx0, rows0, semg0,
                      semi0)

        plsc.subcore_barrier()

        @pl.loop(0, NRLOOP)
        def _wout(kk):
            c = sid + NS * kk

            @pl.when(c < NRC)
            def _():
                pltpu.sync_copy(acc.at[pl.ds(c * RC, RC)],
                                out.at[cid, pl.ds(c * RC, RC)])
                if with_deg:
                    pltpu.sync_copy(dega.at[pl.ds(c * RC, RC)],
                                    degout.at[cid, pl.ds(c * RC, RC)])

    return k


_sc_agg_deg = _make_sc_agg(H, True)
_sc_agg_h = _make_sc_agg(H, False)
_sc_agg_c = _make_sc_agg(C, False)

BN = 1000


def _dot_t(a, w):
    return lax.dot_general(a, w, (((1,), (1,)), ((), ())),
                           precision=lax.Precision.HIGHEST,
                           preferred_element_type=_F32)


def _tc_first_body(x_ref, wl_ref, wr_ref, bl_ref, u_ref, v_ref):
    x = x_ref[...]
    u_ref[...] = _dot_t(x, wl_ref[...])
    v_ref[...] = _dot_t(x, wr_ref[...]) + bl_ref[...]


def _tc_first(x, wl, wr, bl):
    dout = wl.shape[0]
    return pl.pallas_call(
        _tc_first_body,
        grid=(N // BN,),
        in_specs=[
            pl.BlockSpec((BN, x.shape[1]), lambda i: (i, 0)),
            pl.BlockSpec(wl.shape, lambda i: (0, 0)),
            pl.BlockSpec(wr.shape, lambda i: (0, 0)),
            pl.BlockSpec((1, dout), lambda i: (0, 0)),
        ],
        out_specs=[
            pl.BlockSpec((BN, dout), lambda i: (i, 0)),
            pl.BlockSpec((BN, dout), lambda i: (i, 0)),
        ],
        out_shape=[
            jax.ShapeDtypeStruct((N, dout), _F32),
            jax.ShapeDtypeStruct((N, dout), _F32),
        ],
    )(x, wl, wr, bl)


def _tc_mid_body(sa_ref, sb_ref, da_ref, db_ref, v_ref, wl_ref, wr_ref,
                 bl_ref, u_ref, vn_ref):
    deg = da_ref[...] + db_ref[...]
    rdeg = 1.0 / jnp.maximum(deg, 1.0)
    h = jnp.maximum((sa_ref[...] + sb_ref[...]) * rdeg + v_ref[...], 0.0)
    u_ref[...] = _dot_t(h, wl_ref[...])
    vn_ref[...] = _dot_t(h, wr_ref[...]) + bl_ref[...]


def _tc_mid(sa, sb, da, db, v, wl, wr, bl):
    din = sa.shape[1]
    dout = wl.shape[0]
    return pl.pallas_call(
        _tc_mid_body,
        grid=(N // BN,),
        in_specs=[
            pl.BlockSpec((BN, din), lambda i: (i, 0)),
            pl.BlockSpec((BN, din), lambda i: (i, 0)),
            pl.BlockSpec((BN, 1), lambda i: (i, 0)),
            pl.BlockSpec((BN, 1), lambda i: (i, 0)),
            pl.BlockSpec((BN, din), lambda i: (i, 0)),
            pl.BlockSpec(wl.shape, lambda i: (0, 0)),
            pl.BlockSpec(wr.shape, lambda i: (0, 0)),
            pl.BlockSpec((1, dout), lambda i: (0, 0)),
        ],
        out_specs=[
            pl.BlockSpec((BN, dout), lambda i: (i, 0)),
            pl.BlockSpec((BN, dout), lambda i: (i, 0)),
        ],
        out_shape=[
            jax.ShapeDtypeStruct((N, dout), _F32),
            jax.ShapeDtypeStruct((N, dout), _F32),
        ],
    )(sa, sb, da, db, v, wl, wr, bl)


def _tc_final_body(sa_ref, sb_ref, da_ref, db_ref, v_ref, o_ref):
    deg = da_ref[...] + db_ref[...]
    rdeg = 1.0 / jnp.maximum(deg, 1.0)
    o_ref[...] = (sa_ref[...] + sb_ref[...]) * rdeg + v_ref[...]


def _tc_final(sa, sb, da, db, v):
    dout = sa.shape[1]
    return pl.pallas_call(
        _tc_final_body,
        grid=(N // BN,),
        in_specs=[
            pl.BlockSpec((BN, dout), lambda i: (i, 0)),
            pl.BlockSpec((BN, dout), lambda i: (i, 0)),
            pl.BlockSpec((BN, 1), lambda i: (i, 0)),
            pl.BlockSpec((BN, 1), lambda i: (i, 0)),
            pl.BlockSpec((BN, dout), lambda i: (i, 0)),
        ],
        out_specs=pl.BlockSpec((BN, dout), lambda i: (i, 0)),
        out_shape=jax.ShapeDtypeStruct((N, dout), _F32),
    )(sa, sb, da, db, v)


def kernel(x, edge_index, Wl0, bl0, Wr0, Wl1, bl1, Wr1, Wl2, bl2, Wr2):
    eidx = jnp.stack([edge_index[0].reshape(NW, KI, KB),
                      edge_index[1].reshape(NW, KI, KB)], axis=2)
    zrows_h = jnp.zeros((RC, H), _F32)
    zrows_c = jnp.zeros((RC, C), _F32)
    zvec = jnp.zeros((RC, DW), _F32)
    ones_e = jnp.ones((KB, DW), _F32)

    u0, v0 = _tc_first(x, Wl0, Wr0, bl0.reshape(1, -1))
    s0, deg2 = _sc_agg_deg(u0, eidx, zrows_h, zvec, ones_e)
    da, db = deg2[0, :, :1], deg2[1, :, :1]
    u1, v1 = _tc_mid(s0[0], s0[1], da, db, v0, Wl1, Wr1, bl1.reshape(1, -1))
    (s1,) = _sc_agg_h(u1, eidx, zrows_h, zvec, ones_e)
    u2, v2 = _tc_mid(s1[0], s1[1], da, db, v1, Wl2, Wr2, bl2.reshape(1, -1))
    (s2,) = _sc_agg_c(u2, eidx, zrows_c, zvec, ones_e)
    return _tc_final(s2[0], s2[1], da, db, v2)


# speedup vs baseline: 10.6349x; 1.1119x over previous
import functools

import jax
import jax.numpy as jnp
from jax import lax
from jax.experimental import pallas as pl
from jax.experimental.pallas import tpu as pltpu
from jax.experimental.pallas import tpu_sc as plsc

N = 10000
E = 320000
D = 128
H = 128
C = 64

NC, NS = 2, 16
NW = NC * NS
EW = E // NW
KB = 125
KI = EW // KB
RC = 80
NRC = N // RC
NRLOOP = -(-NRC // NS)
DW = 16

_F32 = jnp.float32


def _make_sc_agg(d, with_deg):
    mesh = plsc.VectorSubcoreMesh(core_axis_name="c", subcore_axis_name="s")
    out_type = [jax.ShapeDtypeStruct((NC, N, d), _F32)]
    scratch = (
        [pltpu.VMEM_SHARED((N, d), _F32)]
        + [pltpu.VMEM((2, KB), jnp.int32)] * 4
        + [pltpu.VMEM((KB, d), _F32)] * 2
        + [pltpu.SemaphoreType.DMA] * 2
        + [pltpu.SemaphoreType.DMA] * 4
        + [pltpu.SemaphoreType.DMA] * 2
    )
    if with_deg:
        out_type.append(jax.ShapeDtypeStruct((NC, N, DW), _F32))
        scratch += [
            pltpu.VMEM_SHARED((N, DW), _F32),
            pltpu.VMEM((KB, DW), _F32),
        ]

    @functools.partial(pl.kernel, out_type=tuple(out_type), mesh=mesh,
                       scratch_types=scratch,
                       compiler_params=pltpu.CompilerParams(
                           use_tc_tiling_on_sc=False))
    def k(u_hbm, eidx_hbm, zrows_hbm, zvec_hbm, ones_hbm, *refs):
        if with_deg:
            out, degout = refs[0], refs[1]
            acc = refs[2]
            idx = refs[3:7]
            rows = refs[7:9]
            semg = refs[9:11]
            semi = refs[11:15]
            sems = refs[15:17]
            dega, onesv = refs[17], refs[18]
        else:
            out = refs[0]
            acc = refs[1]
            idx = refs[2:6]
            rows = refs[6:8]
            semg = refs[8:10]
            semi = refs[10:14]
            sems = refs[14:16]
        cid = lax.axis_index("c")
        sid = lax.axis_index("s")
        wid = sid * NC + cid

        if with_deg:
            pltpu.sync_copy(ones_hbm, onesv)

        @pl.loop(0, NRLOOP)
        def _zero(kk):
            c = sid + NS * kk

            @pl.when(c < NRC)
            def _():
                pltpu.sync_copy(zrows_hbm, acc.at[pl.ds(c * RC, RC)])
                if with_deg:
                    pltpu.sync_copy(zvec_hbm, dega.at[pl.ds(c * RC, RC)])

        plsc.subcore_barrier()

        def _load_panel(j, kc, sem=None):
            if sem is None:
                pltpu.sync_copy(eidx_hbm.at[0, wid, kc], idx[j].at[0])
                pltpu.sync_copy(eidx_hbm.at[1, wid, kc], idx[j].at[1])
            else:
                pltpu.async_copy(eidx_hbm.at[0, wid, kc], idx[j].at[0], sem)
                pltpu.async_copy(eidx_hbm.at[1, wid, kc], idx[j].at[1], sem)

        def _drain_panel(j, sem):
            pltpu.make_async_copy(eidx_hbm.at[0, wid, 0], idx[j].at[0],
                                  sem).wait()
            pltpu.make_async_copy(eidx_hbm.at[1, wid, 0], idx[j].at[1],
                                  sem).wait()

        _load_panel(0, 0)
        _load_panel(1, 1, semi[1])
        _load_panel(2, 2, semi[2])
        pltpu.async_copy(u_hbm.at[idx[0].at[0]], rows[0], semg[0])

        def _step(kc, ri, ii):
            rb, ib, ip = 1 - ri, (ii + 1) % 4, (ii + 3) % 4
            pltpu.make_async_copy(u_hbm.at[idx[ii].at[0]], rows[ri],
                                  semg[ri]).wait()

            @pl.when(kc >= 1)
            def _():
                pltpu.make_async_copy(rows[rb], acc.at[idx[ip].at[1]],
                                      sems[rb]).wait()

            @pl.when(kc + 1 < KI)
            def _():
                _drain_panel(ib, semi[ib])
                pltpu.async_copy(u_hbm.at[idx[ib].at[0]], rows[rb], semg[rb])

            pltpu.async_copy(rows[ri], acc.at[idx[ii].at[1]], sems[ri],
                             add=True)
            if with_deg:
                pltpu.sync_copy(onesv, dega.at[idx[ii].at[1]], add=True)

            @pl.when(kc + 3 < KI)
            def _():
                _load_panel(ip, kc + 3, semi[ip])

        @pl.loop(0, KI // 4)
        def _edges(kk):
            k0 = 4 * kk
            for j in range(4):
                _step(k0 + j, j % 2, j)

        pltpu.make_async_copy(rows[(KI - 1) % 2], acc.at[idx[(KI - 1) % 4].at[1]],
                              sems[(KI - 1) % 2]).wait()
        plsc.subcore_barrier()

        @pl.loop(0, NRLOOP)
        def _wout(kk):
            c = sid + NS * kk

            @pl.when(c < NRC)
            def _():
                pltpu.sync_copy(acc.at[pl.ds(c * RC, RC)],
                                out.at[cid, pl.ds(c * RC, RC)])
                if with_deg:
                    pltpu.sync_copy(dega.at[pl.ds(c * RC, RC)],
                                    degout.at[cid, pl.ds(c * RC, RC)])

    return k


_sc_agg_deg = _make_sc_agg(H, True)
_sc_agg_h = _make_sc_agg(H, False)
_sc_agg_c = _make_sc_agg(C, False)

BN = 2000


def _dot_t(a, w):
    return lax.dot_general(a, w, (((1,), (1,)), ((), ())),
                           precision=lax.Precision.HIGHEST,
                           preferred_element_type=_F32)


def _tc_first_body(x_ref, wl_ref, wr_ref, bl_ref, u_ref, v_ref):
    x = x_ref[...]
    u_ref[...] = _dot_t(x, wl_ref[...])
    v_ref[...] = _dot_t(x, wr_ref[...]) + bl_ref[...]


def _tc_first(x, wl, wr, bl):
    dout = wl.shape[0]
    return pl.pallas_call(
        _tc_first_body,
        grid=(N // BN,),
        in_specs=[
            pl.BlockSpec((BN, x.shape[1]), lambda i: (i, 0)),
            pl.BlockSpec(wl.shape, lambda i: (0, 0)),
            pl.BlockSpec(wr.shape, lambda i: (0, 0)),
            pl.BlockSpec((dout,), lambda i: (0,)),
        ],
        out_specs=[
            pl.BlockSpec((BN, dout), lambda i: (i, 0)),
            pl.BlockSpec((BN, dout), lambda i: (i, 0)),
        ],
        out_shape=[
            jax.ShapeDtypeStruct((N, dout), _F32),
            jax.ShapeDtypeStruct((N, dout), _F32),
        ],
    )(x, wl, wr, bl)


def _tc_mid_body(s_ref, deg_ref, v_ref, wl_ref, wr_ref,
                 bl_ref, u_ref, vn_ref):
    deg = deg_ref[0, :, 0:1] + deg_ref[1, :, 0:1]
    rdeg = 1.0 / jnp.maximum(deg, 1.0)
    h = jnp.maximum((s_ref[0] + s_ref[1]) * rdeg + v_ref[...], 0.0)
    u_ref[...] = _dot_t(h, wl_ref[...])
    vn_ref[...] = _dot_t(h, wr_ref[...]) + bl_ref[...]


def _tc_mid(s, deg2, v, wl, wr, bl):
    din = s.shape[2]
    dout = wl.shape[0]
    return pl.pallas_call(
        _tc_mid_body,
        grid=(N // BN,),
        in_specs=[
            pl.BlockSpec((2, BN, din), lambda i: (0, i, 0)),
            pl.BlockSpec((2, BN, DW), lambda i: (0, i, 0)),
            pl.BlockSpec((BN, din), lambda i: (i, 0)),
            pl.BlockSpec(wl.shape, lambda i: (0, 0)),
            pl.BlockSpec(wr.shape, lambda i: (0, 0)),
            pl.BlockSpec((dout,), lambda i: (0,)),
        ],
        out_specs=[
            pl.BlockSpec((BN, dout), lambda i: (i, 0)),
            pl.BlockSpec((BN, dout), lambda i: (i, 0)),
        ],
        out_shape=[
            jax.ShapeDtypeStruct((N, dout), _F32),
            jax.ShapeDtypeStruct((N, dout), _F32),
        ],
    )(s, deg2, v, wl, wr, bl)


def _tc_final_body(s_ref, deg_ref, v_ref, o_ref):
    deg = deg_ref[0, :, 0:1] + deg_ref[1, :, 0:1]
    rdeg = 1.0 / jnp.maximum(deg, 1.0)
    o_ref[...] = (s_ref[0] + s_ref[1]) * rdeg + v_ref[...]


def _tc_final(s, deg2, v):
    dout = s.shape[2]
    return pl.pallas_call(
        _tc_final_body,
        grid=(N // BN,),
        in_specs=[
            pl.BlockSpec((2, BN, dout), lambda i: (0, i, 0)),
            pl.BlockSpec((2, BN, DW), lambda i: (0, i, 0)),
            pl.BlockSpec((BN, dout), lambda i: (i, 0)),
        ],
        out_specs=pl.BlockSpec((BN, dout), lambda i: (i, 0)),
        out_shape=jax.ShapeDtypeStruct((N, dout), _F32),
    )(s, deg2, v)


def kernel(x, edge_index, Wl0, bl0, Wr0, Wl1, bl1, Wr1, Wl2, bl2, Wr2):
    eidx = edge_index.reshape(2, NW, KI, KB)
    zrows_h = jnp.zeros((RC, H), _F32)
    zrows_c = jnp.zeros((RC, C), _F32)
    zvec = jnp.zeros((RC, DW), _F32)
    ones_e = jnp.ones((KB, DW), _F32)

    u0, v0 = _tc_first(x, Wl0, Wr0, bl0)
    s0, deg2 = _sc_agg_deg(u0, eidx, zrows_h, zvec, ones_e)
    u1, v1 = _tc_mid(s0, deg2, v0, Wl1, Wr1, bl1)
    (s1,) = _sc_agg_h(u1, eidx, zrows_h, zvec, ones_e)
    u2, v2 = _tc_mid(s1, deg2, v1, Wl2, Wr2, bl2)
    (s2,) = _sc_agg_c(u2, eidx, zrows_c, zvec, ones_e)
    return _tc_final(s2, deg2, v2)
